# baseline (speedup 1.0000x reference)
import functools
import jax
import jax.numpy as jnp
from jax import lax
from jax.experimental import pallas as pl
from jax.experimental.pallas import tpu as pltpu
from jax.experimental.pallas import tpu_sc as plsc

NUM_WORKERS = 32
LANES = 16
K = 8


@functools.lru_cache(maxsize=None)
def _make_sc_kernel(T: int, N: int, C: int, J: int):
    assert N == NUM_WORKERS
    CH = C * J
    NSL = CH // LANES
    NG = T // K

    mesh = plsc.VectorSubcoreMesh(core_axis_name="c", subcore_axis_name="s")

    @functools.partial(
        pl.kernel,
        out_type=jax.ShapeDtypeStruct((T, N, C, J), jnp.float32),
        mesh=mesh,
        scratch_types=[
            pltpu.VMEM((K, C, J), jnp.float32),
            pltpu.VMEM((K, C, J), jnp.float32),
            pltpu.VMEM((K, C, J), jnp.float32),
            pltpu.VMEM((K, C, J), jnp.float32),
            pltpu.VMEM((J,), jnp.float32),
            pltpu.VMEM((CH,), jnp.float32),
            pltpu.VMEM((CH,), jnp.float32),
            pltpu.SemaphoreType.DMA,
            pltpu.SemaphoreType.DMA,
            pltpu.SemaphoreType.DMA,
            pltpu.SemaphoreType.DMA,
        ],
    )
    def spike_sc(x_hbm, w_hbm, out_hbm, xb0, xb1, yb0, yb1, wv, vv, hv,
                 si0, si1, so0, so1):
        cid = lax.axis_index("c")
        sid = lax.axis_index("s")
        n = sid * 2 + cid

        xbufs = [xb0, xb1]
        ybufs = [yb0, yb1]
        sins = [si0, si1]
        souts = [so0, so1]

        din = [None] * NG
        dout = [None] * NG
        din[0] = pltpu.async_copy(x_hbm.at[pl.ds(0, K), n], xb0, si0)
        din[1] = pltpu.async_copy(x_hbm.at[pl.ds(K, K), n], xb1, si1)

        pltpu.sync_copy(w_hbm, wv)

        def cs(i):
            if C == 1:
                return 0, pl.ds(i * LANES, LANES)
            return (i * LANES) // J, pl.ds((i * LANES) % J, LANES)

        @plsc.parallel_loop(0, NSL, unroll=2)
        def _init(i):
            c, s = cs(i)
            z = jnp.zeros((LANES,), jnp.float32)
            vv[pl.ds(i * LANES, LANES)] = z
            hv[pl.ds(i * LANES, LANES)] = z
            yb1[K - 1, c, s] = z

        for g in range(NG):
            b = g & 1
            xb = xbufs[b]
            yb = ybufs[b]
            ypb = ybufs[1 - b]
            din[g].wait()
            if g >= 2:
                dout[g - 2].wait()

            @plsc.parallel_loop(0, NSL, unroll=2)
            def _block(i, xb=xb, yb=yb, ypb=ypb):
                c, s = cs(i)
                sf = pl.ds(i * LANES, LANES)
                v = vv[sf]
                h = hv[sf]
                w = wv[s if C == 1 else pl.ds((i * LANES) % J, LANES)]
                m = ypb[K - 1, c, s] >= 0.5
                for k in range(K):
                    h = jnp.where(m, h + w, 0.0)
                    v = v + (xb[k, c, s] + h)
                    m = v >= 1.0
                    yb[k, c, s] = jnp.where(m, 1.0, 0.0)
                    v = jnp.where(m, 0.0, v)
                vv[sf] = v
                hv[sf] = h

            dout[g] = pltpu.async_copy(
                yb, out_hbm.at[pl.ds(g * K, K), n], souts[b])
            if g + 2 < NG:
                din[g + 2] = pltpu.async_copy(
                    x_hbm.at[pl.ds((g + 2) * K, K), n], xb, sins[b])

        dout[NG - 2].wait()
        dout[NG - 1].wait()

    return spike_sc


def kernel(input, lateral_weight):
    T, N, C, J = input.shape
    return _make_sc_kernel(T, N, C, J)(input, lateral_weight)


# speedup vs baseline: 1.0677x; 1.0677x over previous
import functools
import jax
import jax.numpy as jnp
from jax import lax
from jax.experimental import pallas as pl
from jax.experimental.pallas import tpu as pltpu
from jax.experimental.pallas import tpu_sc as plsc

NUM_WORKERS = 32
LANES = 16
K = 8
NGB = 4


@functools.lru_cache(maxsize=None)
def _make_sc_kernel(T: int, N: int, C: int, J: int):
    assert N == NUM_WORKERS
    CH = C * J
    NSL = CH // LANES
    NG = T // K
    NLP = NG // NGB

    mesh = plsc.VectorSubcoreMesh(core_axis_name="c", subcore_axis_name="s")

    @functools.partial(
        pl.kernel,
        out_type=jax.ShapeDtypeStruct((T, N, C, J), jnp.float32),
        mesh=mesh,
        scratch_types=[
            pltpu.VMEM((K, C, J), jnp.float32),
            pltpu.VMEM((K, C, J), jnp.float32),
            pltpu.VMEM((K, C, J), jnp.float32),
            pltpu.VMEM((K, C, J), jnp.float32),
            pltpu.VMEM((K, C, J), jnp.float32),
            pltpu.VMEM((K, C, J), jnp.float32),
            pltpu.VMEM((J,), jnp.float32),
            pltpu.VMEM((CH,), jnp.float32),
            pltpu.VMEM((CH,), jnp.float32),
            pltpu.SemaphoreType.DMA,
            pltpu.SemaphoreType.DMA,
            pltpu.SemaphoreType.DMA,
            pltpu.SemaphoreType.DMA,
            pltpu.SemaphoreType.DMA,
            pltpu.SemaphoreType.DMA,
        ],
    )
    def spike_sc(x_hbm, w_hbm, out_hbm, xb0, xb1, yb0, yb1, yb2, yb3,
                 wv, vv, hv, si0, si1, so0, so1, so2, so3):
        cid = lax.axis_index("c")
        sid = lax.axis_index("s")
        n = sid * 2 + cid

        xbufs = [xb0, xb1]
        ybufs = [yb0, yb1, yb2, yb3]
        sins = [si0, si1]
        souts = [so0, so1, so2, so3]

        pltpu.async_copy(x_hbm.at[pl.ds(0, K), n], xb0, si0)
        pltpu.async_copy(x_hbm.at[pl.ds(K, K), n], xb1, si1)

        pltpu.sync_copy(w_hbm, wv)

        def cs(i):
            if C == 1:
                return 0, pl.ds(i * LANES, LANES)
            return (i * LANES) // J, pl.ds((i * LANES) % J, LANES)

        @plsc.parallel_loop(0, NSL, unroll=2)
        def _init(i):
            c, s = cs(i)
            z = jnp.zeros((LANES,), jnp.float32)
            vv[pl.ds(i * LANES, LANES)] = z
            hv[pl.ds(i * LANES, LANES)] = z
            yb3[K - 1, c, s] = z

        def pair_body(gp, carry):
            t0 = gp * (NGB * K)
            for j in range(NGB):
                xb = xbufs[j % 2]
                yb = ybufs[j]
                ypb = ybufs[(j - 1) % NGB]
                sin = sins[j % 2]
                sout = souts[j]
                pltpu.make_async_copy(x_hbm.at[pl.ds(0, K), n], xb,
                                      sin).wait()

                @pl.when(gp >= 1)
                def _wait_out(yb=yb, sout=sout):
                    pltpu.make_async_copy(
                        yb, out_hbm.at[pl.ds(0, K), n], sout).wait()

                @plsc.parallel_loop(0, NSL, unroll=2)
                def _block(i, xb=xb, yb=yb, ypb=ypb):
                    c, s = cs(i)
                    sf = pl.ds(i * LANES, LANES)
                    v = vv[sf]
                    h = hv[sf]
                    w = wv[s if C == 1 else pl.ds((i * LANES) % J, LANES)]
                    m = ypb[K - 1, c, s] >= 0.5
                    for k in range(K):
                        h = jnp.where(m, h + w, 0.0)
                        v = v + (xb[k, c, s] + h)
                        m = v >= 1.0
                        yb[k, c, s] = jnp.where(m, 1.0, 0.0)
                        v = jnp.where(m, 0.0, v)
                    vv[sf] = v
                    hv[sf] = h

                pltpu.async_copy(
                    yb, out_hbm.at[pl.ds(t0 + j * K, K), n], sout)
                if j < 2:
                    pltpu.async_copy(
                        x_hbm.at[pl.ds(t0 + (j + 2) * K, K), n], xb, sin)
                else:
                    @pl.when(gp + 1 < NLP)
                    def _start_in(xb=xb, sin=sin, off=(j + 2) * K):
                        pltpu.async_copy(
                            x_hbm.at[pl.ds(t0 + off, K), n], xb, sin)
            return carry

        lax.fori_loop(0, NLP, pair_body, 0)

        for j in range(NGB):
            pltpu.make_async_copy(
                ybufs[j], out_hbm.at[pl.ds(0, K), n], souts[j]).wait()

    return spike_sc


def kernel(input, lateral_weight):
    T, N, C, J = input.shape
    return _make_sc_kernel(T, N, C, J)(input, lateral_weight)


# speedup vs baseline: 1.1161x; 1.0453x over previous
import functools
import jax
import jax.numpy as jnp
from jax import lax
from jax.experimental import pallas as pl
from jax.experimental.pallas import tpu as pltpu
from jax.experimental.pallas import tpu_sc as plsc

NUM_WORKERS = 32
LANES = 16
K = 8
NGB = 2


@functools.lru_cache(maxsize=None)
def _make_sc_kernel(T: int, N: int, C: int, J: int):
    assert N == NUM_WORKERS
    CH = C * J
    NSL = CH // LANES
    NG = T // K
    NLP = NG // NGB

    mesh = plsc.VectorSubcoreMesh(core_axis_name="c", subcore_axis_name="s")

    @functools.partial(
        pl.kernel,
        out_type=jax.ShapeDtypeStruct((T, N, C, J), jnp.float32),
        mesh=mesh,
        scratch_types=[
            pltpu.VMEM((K, C, J), jnp.float32),
            pltpu.VMEM((K, C, J), jnp.float32),
            pltpu.VMEM((K, C, J), jnp.float32),
            pltpu.VMEM((K, C, J), jnp.float32),
            pltpu.VMEM((J,), jnp.float32),
            pltpu.VMEM((CH,), jnp.float32),
            pltpu.VMEM((CH,), jnp.float32),
            pltpu.SemaphoreType.DMA,
            pltpu.SemaphoreType.DMA,
            pltpu.SemaphoreType.DMA,
            pltpu.SemaphoreType.DMA,
        ],
    )
    def spike_sc(x_hbm, w_hbm, out_hbm, xb0, xb1, yb0, yb1,
                 wv, vv, hv, si0, si1, so0, so1):
        cid = lax.axis_index("c")
        sid = lax.axis_index("s")
        n = sid * 2 + cid

        xbufs = [xb0, xb1]
        ybufs = [yb0, yb1]
        sins = [si0, si1]
        souts = [so0, so1]

        pltpu.async_copy(x_hbm.at[pl.ds(0, K), n], xb0, si0)
        pltpu.async_copy(x_hbm.at[pl.ds(K, K), n], xb1, si1)

        pltpu.sync_copy(w_hbm, wv)

        def cs(i):
            if C == 1:
                return 0, pl.ds(i * LANES, LANES)
            return (i * LANES) // J, pl.ds((i * LANES) % J, LANES)

        @plsc.parallel_loop(0, NSL, unroll=2)
        def _init(i):
            c, s = cs(i)
            z = jnp.zeros((LANES,), jnp.float32)
            vv[pl.ds(i * LANES, LANES)] = z
            hv[pl.ds(i * LANES, LANES)] = z
            yb1[K - 1, c, s] = z

        def pair_body(gp, carry):
            t0 = gp * (NGB * K)
            for j in range(NGB):
                xb = xbufs[j % 2]
                yb = ybufs[j]
                ypb = ybufs[(j - 1) % NGB]
                sin = sins[j % 2]
                sout = souts[j]
                pltpu.make_async_copy(x_hbm.at[pl.ds(0, K), n], xb,
                                      sin).wait()

                @pl.when(gp >= 1)
                def _wait_out(yb=yb, sout=sout):
                    pltpu.make_async_copy(
                        yb, out_hbm.at[pl.ds(0, K), n], sout).wait()

                @plsc.parallel_loop(0, NSL, unroll=2)
                def _block(i, xb=xb, yb=yb, ypb=ypb):
                    c, s = cs(i)
                    sf = pl.ds(i * LANES, LANES)
                    v = vv[sf]
                    h = hv[sf]
                    w = wv[s if C == 1 else pl.ds((i * LANES) % J, LANES)]
                    m = ypb[K - 1, c, s] >= 0.5
                    for k in range(K):
                        h = jnp.where(m, h + w, 0.0)
                        v = v + (xb[k, c, s] + h)
                        m = v >= 1.0
                        yb[k, c, s] = jnp.where(m, 1.0, 0.0)
                        v = jnp.where(m, 0.0, v)
                    vv[sf] = v
                    hv[sf] = h

                pltpu.async_copy(
                    yb, out_hbm.at[pl.ds(t0 + j * K, K), n], sout)

                @pl.when(gp + 1 < NLP)
                def _start_in(xb=xb, sin=sin, off=(j + 2) * K):
                    pltpu.async_copy(
                        x_hbm.at[pl.ds(t0 + off, K), n], xb, sin)
            return carry

        lax.fori_loop(0, NLP, pair_body, 0)

        for j in range(NGB):
            pltpu.make_async_copy(
                ybufs[j], out_hbm.at[pl.ds(0, K), n], souts[j]).wait()

    return spike_sc


def kernel(input, lateral_weight):
    T, N, C, J = input.shape
    return _make_sc_kernel(T, N, C, J)(input, lateral_weight)
